# Initial kernel scaffold; baseline (speedup 1.0000x reference)
import jax
import jax.numpy as jnp
from jax.experimental import pallas as pl


def kernel(logits, actions):
    raise NotImplementedError("write your pallas kernel here")


# speedup vs baseline: 3.0079x; 3.0079x over previous
import functools

import jax
import jax.numpy as jnp
from jax.experimental import pallas as pl
from jax.experimental.pallas import tpu as pltpu

_BC = 32768


def _body(a_ref, x_ref, lp_ref, mode_ref, m_ref, s_ref, g_ref, i_ref,
          *, nsteps, ncols, bc):
    j = pl.program_id(0)

    @pl.when(j == 0)
    def _init():
        m_ref[...] = jnp.full_like(m_ref, -jnp.inf)
        s_ref[...] = jnp.zeros_like(s_ref)
        g_ref[...] = jnp.zeros_like(g_ref)
        i_ref[...] = jnp.zeros_like(i_ref)

    x = x_ref[...]
    gi = jax.lax.broadcasted_iota(jnp.int32, x.shape, 1) + j * bc
    valid = gi < ncols
    xm = jnp.where(valid, x, -jnp.inf)

    m_old = m_ref[...]
    bm = jnp.max(xm, axis=1, keepdims=True)
    nm = jnp.maximum(m_old, bm)
    bs = jnp.sum(jnp.exp(xm - nm), axis=1, keepdims=True)
    s_ref[...] = s_ref[...] * jnp.exp(m_old - nm) + bs
    m_ref[...] = nm

    beq = xm == bm
    bidx = jnp.min(jnp.where(beq, gi, jnp.int32(2**30)), axis=1,
                   keepdims=True)
    i_ref[...] = jnp.where(bm > m_old, bidx, i_ref[...])

    a = a_ref[...]
    g_ref[...] = g_ref[...] + jnp.sum(
        jnp.where(gi == a, x, jnp.float32(0.0)), axis=1, keepdims=True)

    @pl.when(j == nsteps - 1)
    def _fin():
        lse = m_ref[...] + jnp.log(s_ref[...])
        lp_ref[...] = g_ref[...] - lse
        mode_ref[...] = i_ref[...]


@jax.jit
def kernel(logits, actions):
    B, C = logits.shape
    nsteps = pl.cdiv(C, _BC)
    lp, mode = pl.pallas_call(
        functools.partial(_body, nsteps=nsteps, ncols=C, bc=_BC),
        grid=(nsteps,),
        in_specs=[
            pl.BlockSpec((B, 1), lambda j: (0, 0)),
            pl.BlockSpec((B, _BC), lambda j: (0, j)),
        ],
        out_specs=[
            pl.BlockSpec((B, 1), lambda j: (0, 0)),
            pl.BlockSpec((B, 1), lambda j: (0, 0)),
        ],
        out_shape=[
            jax.ShapeDtypeStruct((B, 1), jnp.float32),
            jax.ShapeDtypeStruct((B, 1), jnp.int32),
        ],
        scratch_shapes=[
            pltpu.VMEM((B, 1), jnp.float32),
            pltpu.VMEM((B, 1), jnp.float32),
            pltpu.VMEM((B, 1), jnp.float32),
            pltpu.VMEM((B, 1), jnp.int32),
        ],
    )(actions, logits)
    return lp, mode
